# Initial kernel scaffold; baseline (speedup 1.0000x reference)
import jax
import jax.numpy as jnp
from jax.experimental import pallas as pl


def kernel(points1, features1, points2, features2, W1, b1, g1, be1, W2, b2, g2, be2):
    raise NotImplementedError("write your pallas kernel here")


# speedup vs baseline: 13.4494x; 13.4494x over previous
import functools

import jax
import jax.numpy as jnp
from jax.experimental import pallas as pl
from jax.experimental.pallas import tpu as pltpu

_T = 512


def _knn_mm1_kernel(nb, ni, p1_ref, p2_ref, q1_ref, q2_ref, f1_ref, f2_ref,
                    w1a_ref, w1b_ref, y1_ref, ssum_ref, ssq_ref, acc, accsq):
    b = pl.program_id(0)
    i = pl.program_id(1)
    t, n2 = p1_ref.shape[2], p2_ref.shape[2]

    @pl.when((b == 0) & (i == 0))
    def _init():
        acc[...] = jnp.zeros_like(acc)
        accsq[...] = jnp.zeros_like(accsq)

    cc = jax.lax.dot_general(p1_ref[0], p2_ref[0], (((0,), (0,)), ((), ())),
                             preferred_element_type=jnp.float32)
    aug = jax.lax.dot_general(q1_ref[0], q2_ref[0], (((0,), (0,)), ((), ())),
                              preferred_element_type=jnp.float32,
                              precision=jax.lax.Precision.HIGHEST)
    d = jnp.maximum(aug - 2.0 * cc, 0.0)

    lane = jax.lax.broadcasted_iota(jnp.int32, (t, n2), 1)
    dcur = d
    mins, args = [], []
    for _ in range(3):
        m = jnp.min(dcur, axis=1, keepdims=True)
        am = jnp.min(jnp.where(dcur == m, lane, n2), axis=1,
                     keepdims=True)
        mins.append(m)
        args.append(am)
        dcur = jnp.where(lane == am, jnp.float32(jnp.inf), dcur)

    w0 = 1.0 / (mins[0] + 1e-5)
    w1 = 1.0 / (mins[1] + 1e-5)
    w2 = 1.0 / (mins[2] + 1e-5)
    wsum = w0 + w1 + w2
    zero = jnp.float32(0.0)
    s = (jnp.where(lane == args[0], w0 / wsum, zero)
         + jnp.where(lane == args[1], w1 / wsum, zero)
         + jnp.where(lane == args[2], w2 / wsum, zero))

    f2i = jax.lax.dot_general(f2_ref[0], s, (((1,), (1,)), ((), ())),
                              preferred_element_type=jnp.float32,
                              precision=jax.lax.Precision.HIGHEST)
    y = (jnp.dot(w1a_ref[...], f1_ref[0],
                 preferred_element_type=jnp.float32)
         + jnp.dot(w1b_ref[...], f2i.astype(jnp.bfloat16),
                   preferred_element_type=jnp.float32))
    y1_ref[0] = y
    acc[...] += y
    accsq[...] += y * y

    @pl.when((b == nb - 1) & (i == ni - 1))
    def _fin():
        ssum_ref[...] = jnp.broadcast_to(
            jnp.sum(acc[...], axis=1, keepdims=True), ssum_ref.shape)
        ssq_ref[...] = jnp.broadcast_to(
            jnp.sum(accsq[...], axis=1, keepdims=True), ssq_ref.shape)


def _bn_mm2_kernel(nb, ni, y1_ref, w2_ref, sc_ref, sh_ref,
                   y2_ref, ssum_ref, ssq_ref, acc, accsq):
    b = pl.program_id(0)
    i = pl.program_id(1)

    @pl.when((b == 0) & (i == 0))
    def _init():
        acc[...] = jnp.zeros_like(acc)
        accsq[...] = jnp.zeros_like(accsq)

    z = jnp.maximum(y1_ref[0] * sc_ref[:, 0:1] + sh_ref[:, 0:1], 0.0)
    y = jnp.dot(w2_ref[...], z.astype(jnp.bfloat16),
                preferred_element_type=jnp.float32)
    y2_ref[0] = y
    acc[...] += y
    accsq[...] += y * y

    @pl.when((b == nb - 1) & (i == ni - 1))
    def _fin():
        ssum_ref[...] = jnp.broadcast_to(
            jnp.sum(acc[...], axis=1, keepdims=True), ssum_ref.shape)
        ssq_ref[...] = jnp.broadcast_to(
            jnp.sum(accsq[...], axis=1, keepdims=True), ssq_ref.shape)


def _bn_out_kernel(y2_ref, sc_ref, sh_ref, o_ref):
    o_ref[0] = jnp.maximum(y2_ref[0] * sc_ref[:, 0:1] + sh_ref[:, 0:1], 0.0)


def kernel(points1, features1, points2, features2,
           W1, b1, g1, be1, W2, b2, g2, be2):
    B, _, N1 = points1.shape
    N2 = points2.shape[2]
    C1 = features1.shape[1]
    C2 = features2.shape[1]
    D1 = W1.shape[0]
    D2 = W2.shape[0]
    T = _T
    NI = N1 // T
    f32 = jnp.float32

    bf16 = jnp.bfloat16
    sq1 = jnp.sum(points1 * points1, axis=1, keepdims=True)
    sq2 = jnp.sum(points2 * points2, axis=1, keepdims=True)
    p1bf = jnp.concatenate([points1, jnp.zeros((B, 5, N1), f32)],
                           axis=1).astype(bf16)
    p2bf = jnp.concatenate([points2, jnp.zeros((B, 5, N2), f32)],
                           axis=1).astype(bf16)
    q1aug = jnp.concatenate([sq1, jnp.ones((B, 1, N1), f32),
                             jnp.zeros((B, 6, N1), f32)], axis=1)
    q2aug = jnp.concatenate([jnp.ones((B, 1, N2), f32), sq2,
                             jnp.zeros((B, 6, N2), f32)], axis=1)

    f1bf = features1.astype(bf16)
    W1a = W1[:, :C1].astype(bf16)
    W1b = W1[:, C1:].astype(bf16)
    W2bf = W2.astype(bf16)

    grid = (B, NI)
    y1, ssum1, ssq1 = pl.pallas_call(
        functools.partial(_knn_mm1_kernel, B, NI),
        grid=grid,
        in_specs=[
            pl.BlockSpec((1, 8, T), lambda b, i: (b, 0, i)),
            pl.BlockSpec((1, 8, N2), lambda b, i: (b, 0, 0)),
            pl.BlockSpec((1, 8, T), lambda b, i: (b, 0, i)),
            pl.BlockSpec((1, 8, N2), lambda b, i: (b, 0, 0)),
            pl.BlockSpec((1, C1, T), lambda b, i: (b, 0, i)),
            pl.BlockSpec((1, C2, N2), lambda b, i: (b, 0, 0)),
            pl.BlockSpec((D1, C1), lambda b, i: (0, 0)),
            pl.BlockSpec((D1, C2), lambda b, i: (0, 0)),
        ],
        out_specs=[
            pl.BlockSpec((1, D1, T), lambda b, i: (b, 0, i)),
            pl.BlockSpec((D1, 128), lambda b, i: (0, 0)),
            pl.BlockSpec((D1, 128), lambda b, i: (0, 0)),
        ],
        out_shape=[
            jax.ShapeDtypeStruct((B, D1, N1), f32),
            jax.ShapeDtypeStruct((D1, 128), f32),
            jax.ShapeDtypeStruct((D1, 128), f32),
        ],
        scratch_shapes=[
            pltpu.VMEM((D1, T), f32),
            pltpu.VMEM((D1, T), f32),
        ],
    )(p1bf, p2bf, q1aug, q2aug, f1bf, features2, W1a, W1b)

    cnt = f32(B * N1)
    mean1 = ssum1[:, 0] / cnt
    var1 = ssq1[:, 0] / cnt - mean1 * mean1
    scale1 = g1 / jnp.sqrt(var1 + 1e-5)
    shift1 = be1 - mean1 * scale1
    sc1 = jnp.broadcast_to(scale1[:, None], (D1, 128))
    sh1 = jnp.broadcast_to(shift1[:, None], (D1, 128))

    y2, ssum2, ssq2 = pl.pallas_call(
        functools.partial(_bn_mm2_kernel, B, NI),
        grid=grid,
        in_specs=[
            pl.BlockSpec((1, D1, T), lambda b, i: (b, 0, i)),
            pl.BlockSpec((D2, D1), lambda b, i: (0, 0)),
            pl.BlockSpec((D1, 128), lambda b, i: (0, 0)),
            pl.BlockSpec((D1, 128), lambda b, i: (0, 0)),
        ],
        out_specs=[
            pl.BlockSpec((1, D2, T), lambda b, i: (b, 0, i)),
            pl.BlockSpec((D2, 128), lambda b, i: (0, 0)),
            pl.BlockSpec((D2, 128), lambda b, i: (0, 0)),
        ],
        out_shape=[
            jax.ShapeDtypeStruct((B, D2, N1), f32),
            jax.ShapeDtypeStruct((D2, 128), f32),
            jax.ShapeDtypeStruct((D2, 128), f32),
        ],
        scratch_shapes=[
            pltpu.VMEM((D2, T), f32),
            pltpu.VMEM((D2, T), f32),
        ],
    )(y1, W2bf, sc1, sh1)

    mean2 = ssum2[:, 0] / cnt
    var2 = ssq2[:, 0] / cnt - mean2 * mean2
    scale2 = g2 / jnp.sqrt(var2 + 1e-5)
    shift2 = be2 - mean2 * scale2
    sc2 = jnp.broadcast_to(scale2[:, None], (D2, 128))
    sh2 = jnp.broadcast_to(shift2[:, None], (D2, 128))

    out = pl.pallas_call(
        _bn_out_kernel,
        grid=grid,
        in_specs=[
            pl.BlockSpec((1, D2, T), lambda b, i: (b, 0, i)),
            pl.BlockSpec((D2, 128), lambda b, i: (0, 0)),
            pl.BlockSpec((D2, 128), lambda b, i: (0, 0)),
        ],
        out_specs=pl.BlockSpec((1, D2, T), lambda b, i: (b, 0, i)),
        out_shape=jax.ShapeDtypeStruct((B, D2, N1), f32),
    )(y2, sc2, sh2)
    return out


# speedup vs baseline: 16.1262x; 1.1990x over previous
import functools

import jax
import jax.numpy as jnp
from jax.experimental import pallas as pl
from jax.experimental.pallas import tpu as pltpu

_T = 512


def _knn_mm1_kernel(nb, ni, p1_ref, p2_ref, q1_ref, q2_ref, f1_ref, f2_ref,
                    w1a_ref, w1b_ref, y1_ref, ssum_ref, ssq_ref, acc, accsq):
    b = pl.program_id(0)
    i = pl.program_id(1)
    t, n2 = p1_ref.shape[2], p2_ref.shape[2]

    @pl.when((b == 0) & (i == 0))
    def _init():
        acc[...] = jnp.zeros_like(acc)
        accsq[...] = jnp.zeros_like(accsq)

    cc = jax.lax.dot_general(p1_ref[0], p2_ref[0], (((0,), (0,)), ((), ())),
                             preferred_element_type=jnp.float32)
    d = jnp.maximum((q1_ref[0] + q2_ref[0]) - 2.0 * cc, 0.0)

    lane = jax.lax.broadcasted_iota(jnp.int32, (t, n2), 1)
    inf = jnp.float32(jnp.inf)
    zero = jnp.float32(0.0)
    dcur = d
    sacc = jnp.zeros((t, n2), jnp.float32)
    wsum = jnp.zeros((t, 1), jnp.float32)
    for k in range(3):
        m = jnp.min(dcur, axis=1, keepdims=True)
        am = jnp.min(jnp.where(dcur == m, lane, n2), axis=1,
                     keepdims=True)
        eq = lane == am
        wk = 1.0 / (m + 1e-5)
        wsum = wsum + wk
        sacc = sacc + jnp.where(eq, wk, zero)
        if k < 2:
            dcur = jnp.where(eq, inf, dcur)
    s = sacc * (1.0 / wsum)

    f2i = jax.lax.dot_general(f2_ref[0], s, (((1,), (1,)), ((), ())),
                              preferred_element_type=jnp.float32,
                              precision=jax.lax.Precision.HIGHEST)
    y = (jnp.dot(w1a_ref[...], f1_ref[0],
                 preferred_element_type=jnp.float32)
         + jnp.dot(w1b_ref[...], f2i.astype(jnp.bfloat16),
                   preferred_element_type=jnp.float32))
    y1_ref[0] = y
    acc[...] += y
    accsq[...] += y * y

    @pl.when((b == nb - 1) & (i == ni - 1))
    def _fin():
        ssum_ref[...] = jnp.broadcast_to(
            jnp.sum(acc[...], axis=1, keepdims=True), ssum_ref.shape)
        ssq_ref[...] = jnp.broadcast_to(
            jnp.sum(accsq[...], axis=1, keepdims=True), ssq_ref.shape)


def _bn_mm2_kernel(nb, ni, y1_ref, w2_ref, sc_ref, sh_ref,
                   y2_ref, ssum_ref, ssq_ref, acc, accsq):
    b = pl.program_id(0)
    i = pl.program_id(1)

    @pl.when((b == 0) & (i == 0))
    def _init():
        acc[...] = jnp.zeros_like(acc)
        accsq[...] = jnp.zeros_like(accsq)

    z = jnp.maximum(y1_ref[0] * sc_ref[:, 0:1] + sh_ref[:, 0:1], 0.0)
    y = jnp.dot(w2_ref[...], z.astype(jnp.bfloat16),
                preferred_element_type=jnp.float32)
    y2_ref[0] = y
    acc[...] += y
    accsq[...] += y * y

    @pl.when((b == nb - 1) & (i == ni - 1))
    def _fin():
        ssum_ref[...] = jnp.broadcast_to(
            jnp.sum(acc[...], axis=1, keepdims=True), ssum_ref.shape)
        ssq_ref[...] = jnp.broadcast_to(
            jnp.sum(accsq[...], axis=1, keepdims=True), ssq_ref.shape)


def _bn_out_kernel(y2_ref, sc_ref, sh_ref, o_ref):
    o_ref[0] = jnp.maximum(y2_ref[0] * sc_ref[:, 0:1] + sh_ref[:, 0:1], 0.0)


def kernel(points1, features1, points2, features2,
           W1, b1, g1, be1, W2, b2, g2, be2):
    B, _, N1 = points1.shape
    N2 = points2.shape[2]
    C1 = features1.shape[1]
    C2 = features2.shape[1]
    D1 = W1.shape[0]
    D2 = W2.shape[0]
    T = _T
    NI = N1 // T
    f32 = jnp.float32

    bf16 = jnp.bfloat16
    sq1 = jnp.sum(points1 * points1, axis=1, keepdims=True)
    sq2 = jnp.sum(points2 * points2, axis=1, keepdims=True)
    q1col = jnp.transpose(sq1, (0, 2, 1))
    p1bf = jnp.concatenate([points1, jnp.zeros((B, 5, N1), f32)],
                           axis=1).astype(bf16)
    p2bf = jnp.concatenate([points2, jnp.zeros((B, 5, N2), f32)],
                           axis=1).astype(bf16)

    f1bf = features1.astype(bf16)
    W1a = W1[:, :C1].astype(bf16)
    W1b = W1[:, C1:].astype(bf16)
    W2bf = W2.astype(bf16)

    grid = (B, NI)
    y1, ssum1, ssq1 = pl.pallas_call(
        functools.partial(_knn_mm1_kernel, B, NI),
        grid=grid,
        in_specs=[
            pl.BlockSpec((1, 8, T), lambda b, i: (b, 0, i)),
            pl.BlockSpec((1, 8, N2), lambda b, i: (b, 0, 0)),
            pl.BlockSpec((1, T, 1), lambda b, i: (b, i, 0)),
            pl.BlockSpec((1, 1, N2), lambda b, i: (b, 0, 0)),
            pl.BlockSpec((1, C1, T), lambda b, i: (b, 0, i)),
            pl.BlockSpec((1, C2, N2), lambda b, i: (b, 0, 0)),
            pl.BlockSpec((D1, C1), lambda b, i: (0, 0)),
            pl.BlockSpec((D1, C2), lambda b, i: (0, 0)),
        ],
        out_specs=[
            pl.BlockSpec((1, D1, T), lambda b, i: (b, 0, i)),
            pl.BlockSpec((D1, 128), lambda b, i: (0, 0)),
            pl.BlockSpec((D1, 128), lambda b, i: (0, 0)),
        ],
        out_shape=[
            jax.ShapeDtypeStruct((B, D1, N1), f32),
            jax.ShapeDtypeStruct((D1, 128), f32),
            jax.ShapeDtypeStruct((D1, 128), f32),
        ],
        scratch_shapes=[
            pltpu.VMEM((D1, T), f32),
            pltpu.VMEM((D1, T), f32),
        ],
    )(p1bf, p2bf, q1col, sq2, f1bf, features2, W1a, W1b)

    cnt = f32(B * N1)
    mean1 = ssum1[:, 0] / cnt
    var1 = ssq1[:, 0] / cnt - mean1 * mean1
    scale1 = g1 / jnp.sqrt(var1 + 1e-5)
    shift1 = be1 - mean1 * scale1
    sc1 = jnp.broadcast_to(scale1[:, None], (D1, 128))
    sh1 = jnp.broadcast_to(shift1[:, None], (D1, 128))

    y2, ssum2, ssq2 = pl.pallas_call(
        functools.partial(_bn_mm2_kernel, B, NI),
        grid=grid,
        in_specs=[
            pl.BlockSpec((1, D1, T), lambda b, i: (b, 0, i)),
            pl.BlockSpec((D2, D1), lambda b, i: (0, 0)),
            pl.BlockSpec((D1, 128), lambda b, i: (0, 0)),
            pl.BlockSpec((D1, 128), lambda b, i: (0, 0)),
        ],
        out_specs=[
            pl.BlockSpec((1, D2, T), lambda b, i: (b, 0, i)),
            pl.BlockSpec((D2, 128), lambda b, i: (0, 0)),
            pl.BlockSpec((D2, 128), lambda b, i: (0, 0)),
        ],
        out_shape=[
            jax.ShapeDtypeStruct((B, D2, N1), f32),
            jax.ShapeDtypeStruct((D2, 128), f32),
            jax.ShapeDtypeStruct((D2, 128), f32),
        ],
        scratch_shapes=[
            pltpu.VMEM((D2, T), f32),
            pltpu.VMEM((D2, T), f32),
        ],
    )(y1, W2bf, sc1, sh1)

    mean2 = ssum2[:, 0] / cnt
    var2 = ssq2[:, 0] / cnt - mean2 * mean2
    scale2 = g2 / jnp.sqrt(var2 + 1e-5)
    shift2 = be2 - mean2 * scale2
    sc2 = jnp.broadcast_to(scale2[:, None], (D2, 128))
    sh2 = jnp.broadcast_to(shift2[:, None], (D2, 128))

    out = pl.pallas_call(
        _bn_out_kernel,
        grid=grid,
        in_specs=[
            pl.BlockSpec((1, D2, T), lambda b, i: (b, 0, i)),
            pl.BlockSpec((D2, 128), lambda b, i: (0, 0)),
            pl.BlockSpec((D2, 128), lambda b, i: (0, 0)),
        ],
        out_specs=pl.BlockSpec((1, D2, T), lambda b, i: (b, 0, i)),
        out_shape=jax.ShapeDtypeStruct((B, D2, N1), f32),
    )(y2, sc2, sh2)
    return out
